# baseline (speedup 1.0000x reference)
import jax
import jax.numpy as jnp
from jax.experimental import pallas as pl
from jax.experimental.pallas import tpu as pltpu

RADIUS = 4
K_STEPS = 4
MU_MAX = 10.0
ETA = 0.1
_SQRT2 = 1.4142135623730951
PAD = 8
TS = 256


def _softplus(x):
    return jnp.maximum(x, 0.0) + jnp.log(1.0 + jnp.exp(-jnp.abs(x)))


def _gelu(x):
    return 0.5 * x * (jax.lax.erf(x / _SQRT2) + 1.0)


def _body(h_ref, w1a_ref, w1b_ref, b1_ref, w2_ref, b2_ref, wq_ref, bq_ref,
          gamma_ref, beta_ref, out_ref, energy_ref,
          cur_ref, nxt_ref, q_ref, coef_ref):
    S = h_ref.shape[1]
    H = h_ref.shape[2]
    NT = S // TS
    zero_h = jnp.zeros((PAD, H), jnp.float32)

    cur_ref[0:PAD, :] = zero_h
    cur_ref[PAD + S:PAD + S + PAD, :] = zero_h
    nxt_ref[0:PAD, :] = zero_h
    nxt_ref[PAD + S:PAD + S + PAD, :] = zero_h
    coef_ref[0:PAD, :] = jnp.zeros((PAD, 128), jnp.float32)
    coef_ref[PAD + S:PAD + S + PAD, :] = jnp.zeros((PAD, 128), jnp.float32)

    wq = wq_ref[...]
    bq = bq_ref[0]
    b1 = b1_ref[0]
    b2 = b2_ref[0, 0]

    def stage_body(t, _):
        t0 = t * TS
        h_t = h_ref[0, pl.ds(t0, TS), :]
        cur_ref[pl.ds(PAD + t0, TS), :] = h_t
        q_ref[pl.ds(t0, TS), :] = ETA * (
            jnp.dot(h_t, wq, preferred_element_type=jnp.float32) + bq)
        return 0
    jax.lax.fori_loop(0, NT, stage_body, 0)

    def coef_body(t, _):
        t0 = t * TS
        ext = cur_ref[pl.ds(t0, TS + 2 * PAD), :]
        nsq_e = jnp.sum(ext * ext, axis=1, keepdims=True)
        norm_e = jnp.maximum(jnp.sqrt(nsq_e), 1e-6)
        h_t = ext[PAD:PAD + TS]
        nsq_t = nsq_e[PAD:PAD + TS]
        norm_t = norm_e[PAD:PAD + TS]
        row = t0 + jax.lax.broadcasted_iota(jnp.int32, (TS, 1), 0)
        w1mat = jnp.concatenate([w1a_ref[...], w1b_ref[...]], axis=0)
        w2mat = w2_ref[...]
        for d in range(1, RADIUS + 1):
            h_d = ext[PAD + d:PAD + d + TS]
            dot = jnp.sum(h_t * h_d, axis=1, keepdims=True)
            dsq = jnp.maximum(nsq_t + nsq_e[PAD + d:PAD + d + TS] - 2.0 * dot,
                              0.0)
            dist = jnp.sqrt(dsq)
            cos = dot / (norm_t * norm_e[PAD + d:PAD + d + TS])
            feats = jnp.concatenate([dist, cos], axis=1)
            hid = _gelu(
                jnp.dot(feats, w1mat, preferred_element_type=jnp.float32)
                + b1)
            mu = _softplus(
                jnp.dot(hid, w2mat, preferred_element_type=jnp.float32) + b2)
            mu = jnp.minimum(mu + 1e-5, MU_MAX)
            a_d = jnp.where(row < (S - d), mu, 0.0)
            coef_ref[pl.ds(PAD + t0, TS), d - 1:d] = ETA * a_d
            coef_ref[pl.ds(PAD + t0, TS), RADIUS + d - 1:RADIUS + d] = a_d
        return 0
    jax.lax.fori_loop(0, NT, coef_body, 0)

    def cc_body(t, _):
        t0 = t * TS
        cext = coef_ref[pl.ds(t0, TS + 2 * PAD), :]
        etadeg = jnp.zeros((TS, 1), jnp.float32)
        for d in range(1, RADIUS + 1):
            etadeg = etadeg + cext[PAD:PAD + TS, d - 1:d]
            etadeg = etadeg + cext[PAD - d:PAD - d + TS, d - 1:d]
        coef_ref[pl.ds(PAD + t0, TS), 2 * RADIUS:2 * RADIUS + 1] = (
            1.0 - etadeg)
        return 0
    jax.lax.fori_loop(0, NT, cc_body, 0)

    bufs = [cur_ref, nxt_ref]
    for step in range(K_STEPS):
        src = bufs[step % 2]
        dst = bufs[(step + 1) % 2]

        def sweep_body(t, _, src=src, dst=dst):
            t0 = t * TS
            ext = src[pl.ds(t0, TS + 2 * PAD), :]
            cext = coef_ref[pl.ds(t0, TS + 2 * PAD), :]
            s_t = ext[PAD:PAD + TS]
            acc = (cext[PAD:PAD + TS, 2 * RADIUS:2 * RADIUS + 1] * s_t
                   + q_ref[pl.ds(t0, TS), :])
            for d in range(1, RADIUS + 1):
                a_t = cext[PAD:PAD + TS, d - 1:d]
                b_t = cext[PAD - d:PAD - d + TS, d - 1:d]
                acc = acc + a_t * ext[PAD + d:PAD + d + TS]
                acc = acc + b_t * ext[PAD - d:PAD - d + TS]
            dst[pl.ds(PAD + t0, TS), :] = acc
            return 0
        jax.lax.fori_loop(0, NT, sweep_body, 0)

    fin = bufs[K_STEPS % 2]

    def energy_body(t, acc):
        t0 = t * TS
        ext = fin[pl.ds(t0, TS + 2 * PAD), :]
        s_t = ext[PAD:PAD + TS]
        for d in range(1, RADIUS + 1):
            diff = s_t - ext[PAD + d:PAD + d + TS]
            sq = jnp.sum(diff * diff, axis=1, keepdims=True)
            a_t = coef_ref[pl.ds(PAD + t0, TS), RADIUS + d - 1:RADIUS + d]
            acc = acc + jnp.sum(a_t * sq)
        return acc
    energy = jax.lax.fori_loop(0, NT, energy_body, jnp.float32(0.0))
    energy_ref[...] = jnp.full((1, 1, 128), 0.5 * energy, jnp.float32)

    gamma = gamma_ref[0]
    beta = beta_ref[0]

    def ln_body(t, _):
        t0 = t * TS
        pre = fin[pl.ds(PAD + t0, TS), :] + h_ref[0, pl.ds(t0, TS), :]
        mean = jnp.mean(pre, axis=1, keepdims=True)
        cent = pre - mean
        var = jnp.mean(cent * cent, axis=1, keepdims=True)
        out_ref[0, pl.ds(t0, TS), :] = (
            cent / jnp.sqrt(var + 1e-5) * gamma + beta)
        return 0
    jax.lax.fori_loop(0, NT, ln_body, 0)


def kernel(hidden, attention_mask, W1, b1, W2, b2, Wq, bq, gamma, beta):
    B, S, H = hidden.shape
    INNER = W1.shape[1]
    del attention_mask

    w1a = W1[0].reshape(1, INNER)
    w1b = W1[1].reshape(1, INNER)
    b1r = b1.reshape(1, INNER)
    b2r = b2.reshape(1, 1)
    bqr = bq.reshape(1, H)
    gr = gamma.reshape(1, H)
    br = beta.reshape(1, H)

    rep = lambda *_: (0, 0)
    out, energy = pl.pallas_call(
        _body,
        grid=(B,),
        in_specs=[
            pl.BlockSpec((1, S, H), lambda b: (b, 0, 0)),
            pl.BlockSpec((1, INNER), rep),
            pl.BlockSpec((1, INNER), rep),
            pl.BlockSpec((1, INNER), rep),
            pl.BlockSpec((INNER, 1), rep),
            pl.BlockSpec((1, 1), rep),
            pl.BlockSpec((H, H), rep),
            pl.BlockSpec((1, H), rep),
            pl.BlockSpec((1, H), rep),
            pl.BlockSpec((1, H), rep),
        ],
        out_specs=[
            pl.BlockSpec((1, S, H), lambda b: (b, 0, 0)),
            pl.BlockSpec((1, 1, 128), lambda b: (b, 0, 0)),
        ],
        out_shape=[
            jax.ShapeDtypeStruct((B, S, H), jnp.float32),
            jax.ShapeDtypeStruct((B, 1, 128), jnp.float32),
        ],
        scratch_shapes=[
            pltpu.VMEM((S + 2 * PAD, H), jnp.float32),
            pltpu.VMEM((S + 2 * PAD, H), jnp.float32),
            pltpu.VMEM((S, H), jnp.float32),
            pltpu.VMEM((S + 2 * PAD, 128), jnp.float32),
        ],
    )(hidden, w1a, w1b, b1r, W2, b2r, Wq, bqr, gr, br)
    return out, energy[:, 0, 0]


# speedup vs baseline: 1.0421x; 1.0421x over previous
import jax
import jax.numpy as jnp
from jax.experimental import pallas as pl
from jax.experimental.pallas import tpu as pltpu

RADIUS = 4
K_STEPS = 4
MU_MAX = 10.0
ETA = 0.1
_SQRT2 = 1.4142135623730951
PAD = 8
TS = 512


def _softplus(x):
    return jnp.maximum(x, 0.0) + jnp.log(1.0 + jnp.exp(-jnp.abs(x)))


def _gelu(x):
    return 0.5 * x * (jax.lax.erf(x / _SQRT2) + 1.0)


def _body(h_ref, w1a_ref, w1b_ref, b1_ref, w2_ref, b2_ref, wq_ref, bq_ref,
          gamma_ref, beta_ref, out_ref, energy_ref,
          cur_ref, nxt_ref, q_ref, coef_ref):
    S = h_ref.shape[1]
    H = h_ref.shape[2]
    NT = S // TS
    zero_h = jnp.zeros((PAD, H), jnp.float32)

    cur_ref[0:PAD, :] = zero_h
    cur_ref[PAD + S:PAD + S + PAD, :] = zero_h
    nxt_ref[0:PAD, :] = zero_h
    nxt_ref[PAD + S:PAD + S + PAD, :] = zero_h
    coef_ref[0:PAD, :] = jnp.zeros((PAD, 128), jnp.float32)
    coef_ref[PAD + S:PAD + S + PAD, :] = jnp.zeros((PAD, 128), jnp.float32)

    wq = wq_ref[...]
    bq = bq_ref[0]
    b1 = b1_ref[0]
    b2 = b2_ref[0, 0]

    def stage_body(t, _):
        t0 = t * TS
        h_t = h_ref[0, pl.ds(t0, TS), :]
        cur_ref[pl.ds(PAD + t0, TS), :] = h_t
        q_ref[pl.ds(t0, TS), :] = ETA * (
            jnp.dot(h_t, wq, preferred_element_type=jnp.float32) + bq)
        return 0
    jax.lax.fori_loop(0, NT, stage_body, 0)

    def coef_body(t, _):
        t0 = t * TS
        ext = cur_ref[pl.ds(t0, TS + 2 * PAD), :]
        nsq_e = jnp.sum(ext * ext, axis=1, keepdims=True)
        norm_e = jnp.maximum(jnp.sqrt(nsq_e), 1e-6)
        h_t = ext[PAD:PAD + TS]
        nsq_t = nsq_e[PAD:PAD + TS]
        norm_t = norm_e[PAD:PAD + TS]
        row = t0 + jax.lax.broadcasted_iota(jnp.int32, (TS, 1), 0)
        w1mat = jnp.concatenate([w1a_ref[...], w1b_ref[...]], axis=0)
        w2mat = w2_ref[...]
        for d in range(1, RADIUS + 1):
            h_d = ext[PAD + d:PAD + d + TS]
            dot = jnp.sum(h_t * h_d, axis=1, keepdims=True)
            dsq = jnp.maximum(nsq_t + nsq_e[PAD + d:PAD + d + TS] - 2.0 * dot,
                              0.0)
            dist = jnp.sqrt(dsq)
            cos = dot / (norm_t * norm_e[PAD + d:PAD + d + TS])
            feats = jnp.concatenate([dist, cos], axis=1)
            hid = _gelu(
                jnp.dot(feats, w1mat, preferred_element_type=jnp.float32)
                + b1)
            mu = _softplus(
                jnp.dot(hid, w2mat, preferred_element_type=jnp.float32) + b2)
            mu = jnp.minimum(mu + 1e-5, MU_MAX)
            a_d = jnp.where(row < (S - d), mu, 0.0)
            coef_ref[pl.ds(PAD + t0, TS), d - 1:d] = ETA * a_d
            coef_ref[pl.ds(PAD + t0, TS), RADIUS + d - 1:RADIUS + d] = a_d
        return 0
    jax.lax.fori_loop(0, NT, coef_body, 0)

    def cc_body(t, _):
        t0 = t * TS
        cext = coef_ref[pl.ds(t0, TS + 2 * PAD), :]
        etadeg = jnp.zeros((TS, 1), jnp.float32)
        for d in range(1, RADIUS + 1):
            etadeg = etadeg + cext[PAD:PAD + TS, d - 1:d]
            etadeg = etadeg + cext[PAD - d:PAD - d + TS, d - 1:d]
        coef_ref[pl.ds(PAD + t0, TS), 2 * RADIUS:2 * RADIUS + 1] = (
            1.0 - etadeg)
        return 0
    jax.lax.fori_loop(0, NT, cc_body, 0)

    bufs = [cur_ref, nxt_ref]
    for step in range(K_STEPS):
        src = bufs[step % 2]
        dst = bufs[(step + 1) % 2]

        def sweep_body(t, _, src=src, dst=dst):
            t0 = t * TS
            ext = src[pl.ds(t0, TS + 2 * PAD), :]
            cext = coef_ref[pl.ds(t0, TS + 2 * PAD), :]
            s_t = ext[PAD:PAD + TS]
            acc = (cext[PAD:PAD + TS, 2 * RADIUS:2 * RADIUS + 1] * s_t
                   + q_ref[pl.ds(t0, TS), :])
            for d in range(1, RADIUS + 1):
                a_t = cext[PAD:PAD + TS, d - 1:d]
                b_t = cext[PAD - d:PAD - d + TS, d - 1:d]
                acc = acc + a_t * ext[PAD + d:PAD + d + TS]
                acc = acc + b_t * ext[PAD - d:PAD - d + TS]
            dst[pl.ds(PAD + t0, TS), :] = acc
            return 0
        jax.lax.fori_loop(0, NT, sweep_body, 0)

    fin = bufs[K_STEPS % 2]

    def energy_body(t, acc):
        t0 = t * TS
        ext = fin[pl.ds(t0, TS + 2 * PAD), :]
        s_t = ext[PAD:PAD + TS]
        for d in range(1, RADIUS + 1):
            diff = s_t - ext[PAD + d:PAD + d + TS]
            sq = jnp.sum(diff * diff, axis=1, keepdims=True)
            a_t = coef_ref[pl.ds(PAD + t0, TS), RADIUS + d - 1:RADIUS + d]
            acc = acc + jnp.sum(a_t * sq)
        return acc
    energy = jax.lax.fori_loop(0, NT, energy_body, jnp.float32(0.0))
    energy_ref[...] = jnp.full((1, 1, 128), 0.5 * energy, jnp.float32)

    gamma = gamma_ref[0]
    beta = beta_ref[0]

    def ln_body(t, _):
        t0 = t * TS
        pre = fin[pl.ds(PAD + t0, TS), :] + h_ref[0, pl.ds(t0, TS), :]
        mean = jnp.mean(pre, axis=1, keepdims=True)
        cent = pre - mean
        var = jnp.mean(cent * cent, axis=1, keepdims=True)
        out_ref[0, pl.ds(t0, TS), :] = (
            cent / jnp.sqrt(var + 1e-5) * gamma + beta)
        return 0
    jax.lax.fori_loop(0, NT, ln_body, 0)


def kernel(hidden, attention_mask, W1, b1, W2, b2, Wq, bq, gamma, beta):
    B, S, H = hidden.shape
    INNER = W1.shape[1]
    del attention_mask

    w1a = W1[0].reshape(1, INNER)
    w1b = W1[1].reshape(1, INNER)
    b1r = b1.reshape(1, INNER)
    b2r = b2.reshape(1, 1)
    bqr = bq.reshape(1, H)
    gr = gamma.reshape(1, H)
    br = beta.reshape(1, H)

    rep = lambda *_: (0, 0)
    out, energy = pl.pallas_call(
        _body,
        grid=(B,),
        in_specs=[
            pl.BlockSpec((1, S, H), lambda b: (b, 0, 0)),
            pl.BlockSpec((1, INNER), rep),
            pl.BlockSpec((1, INNER), rep),
            pl.BlockSpec((1, INNER), rep),
            pl.BlockSpec((INNER, 1), rep),
            pl.BlockSpec((1, 1), rep),
            pl.BlockSpec((H, H), rep),
            pl.BlockSpec((1, H), rep),
            pl.BlockSpec((1, H), rep),
            pl.BlockSpec((1, H), rep),
        ],
        out_specs=[
            pl.BlockSpec((1, S, H), lambda b: (b, 0, 0)),
            pl.BlockSpec((1, 1, 128), lambda b: (b, 0, 0)),
        ],
        out_shape=[
            jax.ShapeDtypeStruct((B, S, H), jnp.float32),
            jax.ShapeDtypeStruct((B, 1, 128), jnp.float32),
        ],
        scratch_shapes=[
            pltpu.VMEM((S + 2 * PAD, H), jnp.float32),
            pltpu.VMEM((S + 2 * PAD, H), jnp.float32),
            pltpu.VMEM((S, H), jnp.float32),
            pltpu.VMEM((S + 2 * PAD, 128), jnp.float32),
        ],
    )(hidden, w1a, w1b, b1r, W2, b2r, Wq, bqr, gr, br)
    return out, energy[:, 0, 0]


# speedup vs baseline: 1.0730x; 1.0297x over previous
import jax
import jax.numpy as jnp
from jax.experimental import pallas as pl
from jax.experimental.pallas import tpu as pltpu

RADIUS = 4
K_STEPS = 4
MU_MAX = 10.0
ETA = 0.1
_SQRT2 = 1.4142135623730951
PAD = 8
TS = 1024


def _softplus(x):
    return jnp.maximum(x, 0.0) + jnp.log(1.0 + jnp.exp(-jnp.abs(x)))


def _gelu(x):
    return 0.5 * x * (jax.lax.erf(x / _SQRT2) + 1.0)


def _body(h_ref, w1a_ref, w1b_ref, b1_ref, w2_ref, b2_ref, wq_ref, bq_ref,
          gamma_ref, beta_ref, out_ref, energy_ref,
          cur_ref, nxt_ref, q_ref, coef_ref):
    S = h_ref.shape[1]
    H = h_ref.shape[2]
    NT = S // TS
    zero_h = jnp.zeros((PAD, H), jnp.float32)

    cur_ref[0:PAD, :] = zero_h
    cur_ref[PAD + S:PAD + S + PAD, :] = zero_h
    nxt_ref[0:PAD, :] = zero_h
    nxt_ref[PAD + S:PAD + S + PAD, :] = zero_h
    coef_ref[0:PAD, :] = jnp.zeros((PAD, 128), jnp.float32)
    coef_ref[PAD + S:PAD + S + PAD, :] = jnp.zeros((PAD, 128), jnp.float32)

    wq = wq_ref[...]
    bq = bq_ref[0]
    b1 = b1_ref[0]
    b2 = b2_ref[0, 0]

    def stage_body(t, _):
        t0 = t * TS
        h_t = h_ref[0, pl.ds(t0, TS), :]
        cur_ref[pl.ds(PAD + t0, TS), :] = h_t
        q_ref[pl.ds(t0, TS), :] = ETA * (
            jnp.dot(h_t, wq, preferred_element_type=jnp.float32) + bq)
        return 0
    jax.lax.fori_loop(0, NT, stage_body, 0)

    def coef_body(t, _):
        t0 = t * TS
        ext = cur_ref[pl.ds(t0, TS + 2 * PAD), :]
        nsq_e = jnp.sum(ext * ext, axis=1, keepdims=True)
        norm_e = jnp.maximum(jnp.sqrt(nsq_e), 1e-6)
        h_t = ext[PAD:PAD + TS]
        nsq_t = nsq_e[PAD:PAD + TS]
        norm_t = norm_e[PAD:PAD + TS]
        row = t0 + jax.lax.broadcasted_iota(jnp.int32, (TS, 1), 0)
        w1mat = jnp.concatenate([w1a_ref[...], w1b_ref[...]], axis=0)
        w2mat = w2_ref[...]
        for d in range(1, RADIUS + 1):
            h_d = ext[PAD + d:PAD + d + TS]
            dot = jnp.sum(h_t * h_d, axis=1, keepdims=True)
            dsq = jnp.maximum(nsq_t + nsq_e[PAD + d:PAD + d + TS] - 2.0 * dot,
                              0.0)
            dist = jnp.sqrt(dsq)
            cos = dot / (norm_t * norm_e[PAD + d:PAD + d + TS])
            feats = jnp.concatenate([dist, cos], axis=1)
            hid = _gelu(
                jnp.dot(feats, w1mat, preferred_element_type=jnp.float32)
                + b1)
            mu = _softplus(
                jnp.dot(hid, w2mat, preferred_element_type=jnp.float32) + b2)
            mu = jnp.minimum(mu + 1e-5, MU_MAX)
            a_d = jnp.where(row < (S - d), mu, 0.0)
            coef_ref[pl.ds(PAD + t0, TS), d - 1:d] = ETA * a_d
            coef_ref[pl.ds(PAD + t0, TS), RADIUS + d - 1:RADIUS + d] = a_d
        return 0
    jax.lax.fori_loop(0, NT, coef_body, 0)

    def cc_body(t, _):
        t0 = t * TS
        cext = coef_ref[pl.ds(t0, TS + 2 * PAD), :]
        etadeg = jnp.zeros((TS, 1), jnp.float32)
        for d in range(1, RADIUS + 1):
            etadeg = etadeg + cext[PAD:PAD + TS, d - 1:d]
            etadeg = etadeg + cext[PAD - d:PAD - d + TS, d - 1:d]
        coef_ref[pl.ds(PAD + t0, TS), 2 * RADIUS:2 * RADIUS + 1] = (
            1.0 - etadeg)
        return 0
    jax.lax.fori_loop(0, NT, cc_body, 0)

    bufs = [cur_ref, nxt_ref]
    for step in range(K_STEPS):
        src = bufs[step % 2]
        dst = bufs[(step + 1) % 2]

        def sweep_body(t, _, src=src, dst=dst):
            t0 = t * TS
            ext = src[pl.ds(t0, TS + 2 * PAD), :]
            cext = coef_ref[pl.ds(t0, TS + 2 * PAD), :]
            s_t = ext[PAD:PAD + TS]
            acc = (cext[PAD:PAD + TS, 2 * RADIUS:2 * RADIUS + 1] * s_t
                   + q_ref[pl.ds(t0, TS), :])
            for d in range(1, RADIUS + 1):
                a_t = cext[PAD:PAD + TS, d - 1:d]
                b_t = cext[PAD - d:PAD - d + TS, d - 1:d]
                acc = acc + a_t * ext[PAD + d:PAD + d + TS]
                acc = acc + b_t * ext[PAD - d:PAD - d + TS]
            dst[pl.ds(PAD + t0, TS), :] = acc
            return 0
        jax.lax.fori_loop(0, NT, sweep_body, 0)

    fin = bufs[K_STEPS % 2]

    def energy_body(t, acc):
        t0 = t * TS
        ext = fin[pl.ds(t0, TS + 2 * PAD), :]
        s_t = ext[PAD:PAD + TS]
        for d in range(1, RADIUS + 1):
            diff = s_t - ext[PAD + d:PAD + d + TS]
            sq = jnp.sum(diff * diff, axis=1, keepdims=True)
            a_t = coef_ref[pl.ds(PAD + t0, TS), RADIUS + d - 1:RADIUS + d]
            acc = acc + jnp.sum(a_t * sq)
        return acc
    energy = jax.lax.fori_loop(0, NT, energy_body, jnp.float32(0.0))
    energy_ref[...] = jnp.full((1, 1, 128), 0.5 * energy, jnp.float32)

    gamma = gamma_ref[0]
    beta = beta_ref[0]

    def ln_body(t, _):
        t0 = t * TS
        pre = fin[pl.ds(PAD + t0, TS), :] + h_ref[0, pl.ds(t0, TS), :]
        mean = jnp.mean(pre, axis=1, keepdims=True)
        cent = pre - mean
        var = jnp.mean(cent * cent, axis=1, keepdims=True)
        out_ref[0, pl.ds(t0, TS), :] = (
            cent / jnp.sqrt(var + 1e-5) * gamma + beta)
        return 0
    jax.lax.fori_loop(0, NT, ln_body, 0)


def kernel(hidden, attention_mask, W1, b1, W2, b2, Wq, bq, gamma, beta):
    B, S, H = hidden.shape
    INNER = W1.shape[1]
    del attention_mask

    w1a = W1[0].reshape(1, INNER)
    w1b = W1[1].reshape(1, INNER)
    b1r = b1.reshape(1, INNER)
    b2r = b2.reshape(1, 1)
    bqr = bq.reshape(1, H)
    gr = gamma.reshape(1, H)
    br = beta.reshape(1, H)

    rep = lambda *_: (0, 0)
    out, energy = pl.pallas_call(
        _body,
        grid=(B,),
        in_specs=[
            pl.BlockSpec((1, S, H), lambda b: (b, 0, 0)),
            pl.BlockSpec((1, INNER), rep),
            pl.BlockSpec((1, INNER), rep),
            pl.BlockSpec((1, INNER), rep),
            pl.BlockSpec((INNER, 1), rep),
            pl.BlockSpec((1, 1), rep),
            pl.BlockSpec((H, H), rep),
            pl.BlockSpec((1, H), rep),
            pl.BlockSpec((1, H), rep),
            pl.BlockSpec((1, H), rep),
        ],
        out_specs=[
            pl.BlockSpec((1, S, H), lambda b: (b, 0, 0)),
            pl.BlockSpec((1, 1, 128), lambda b: (b, 0, 0)),
        ],
        out_shape=[
            jax.ShapeDtypeStruct((B, S, H), jnp.float32),
            jax.ShapeDtypeStruct((B, 1, 128), jnp.float32),
        ],
        scratch_shapes=[
            pltpu.VMEM((S + 2 * PAD, H), jnp.float32),
            pltpu.VMEM((S + 2 * PAD, H), jnp.float32),
            pltpu.VMEM((S, H), jnp.float32),
            pltpu.VMEM((S + 2 * PAD, 128), jnp.float32),
        ],
    )(hidden, w1a, w1b, b1r, W2, b2r, Wq, bqr, gr, br)
    return out, energy[:, 0, 0]
